# Initial kernel scaffold; baseline (speedup 1.0000x reference)
import jax
import jax.numpy as jnp
from jax.experimental import pallas as pl


def kernel(x_num, x_cat, num_weight, num_bias, emb_table, cat_bias):
    raise NotImplementedError("write your pallas kernel here")


# speedup vs baseline: 1.1715x; 1.1715x over previous
import jax
import jax.numpy as jnp
from jax import lax
from jax.experimental import pallas as pl
from jax.experimental.pallas import tpu as pltpu
from jax.experimental.pallas import tpu_sc as plsc

B = 16384
NF = 100
NC = 26
CARD = 1000
D = 128
TOK = NF + NC

NUM_CORES = 2
NUM_SUBCORES = 16
NW = NUM_CORES * NUM_SUBCORES
ROWS_PER_W = B * NC // NW
CHUNK_R = 128
N_CHUNKS = ROWS_PER_W // CHUNK_R


def _sc_gather_body(table_hbm, idx_hbm, out_hbm, idx_v, rows_v, sem):
    cid = lax.axis_index("c")
    sid = lax.axis_index("s")
    wid = sid * NUM_CORES + cid
    base_r = wid * ROWS_PER_W

    pltpu.sync_copy(idx_hbm.at[wid], idx_v)

    def chunk(g, carry):
        pltpu.async_copy(table_hbm.at[idx_v.at[g]], rows_v, sem).wait()
        pltpu.sync_copy(rows_v, out_hbm.at[pl.ds(base_r + g * CHUNK_R, CHUNK_R)])
        return carry

    lax.fori_loop(0, N_CHUNKS, chunk, 0)


def _sc_gather(emb_table, gidx):
    mesh = plsc.VectorSubcoreMesh(core_axis_name="c", subcore_axis_name="s")
    return pl.kernel(
        _sc_gather_body,
        out_type=jax.ShapeDtypeStruct((B * NC, D), jnp.float32),
        mesh=mesh,
        scratch_types=[
            pltpu.VMEM((N_CHUNKS, CHUNK_R), jnp.int32),
            pltpu.VMEM((CHUNK_R, D), jnp.float32),
            pltpu.SemaphoreType.DMA,
        ],
    )(emb_table, gidx)


BB = 128


def _tc_body(x_ref, w_ref, b_ref, cat_ref, cb_ref, out_ref):
    for f in range(NF):
        xs = x_ref[:, f : f + 1]
        out_ref[:, f * D : (f + 1) * D] = (
            xs * w_ref[f : f + 1, :] + b_ref[f : f + 1, :]
        )
    for f in range(NC):
        out_ref[:, (NF + f) * D : (NF + f + 1) * D] = (
            cat_ref[:, f * D : (f + 1) * D] + cb_ref[f : f + 1, :]
        )


def _tc_assemble(x_num, num_weight, num_bias, cat_rows, cat_bias):
    return pl.pallas_call(
        _tc_body,
        grid=(B // BB,),
        in_specs=[
            pl.BlockSpec((BB, NF), lambda i: (i, 0)),
            pl.BlockSpec((NF, D), lambda i: (0, 0)),
            pl.BlockSpec((NF, D), lambda i: (0, 0)),
            pl.BlockSpec((BB, NC * D), lambda i: (i, 0)),
            pl.BlockSpec((NC, D), lambda i: (0, 0)),
        ],
        out_specs=pl.BlockSpec((BB, TOK * D), lambda i: (i, 0)),
        out_shape=jax.ShapeDtypeStruct((B, TOK * D), jnp.float32),
    )(x_num, num_weight, num_bias, cat_rows, cat_bias)


def kernel(x_num, x_cat, num_weight, num_bias, emb_table, cat_bias):
    offsets = jnp.arange(NC, dtype=jnp.int32) * CARD
    gidx = (x_cat.astype(jnp.int32) + offsets[None, :]).reshape(
        NW, N_CHUNKS, CHUNK_R
    )
    cat_rows = _sc_gather(emb_table, gidx)
    out2d = _tc_assemble(
        x_num, num_weight, num_bias, cat_rows.reshape(B, NC * D), cat_bias
    )
    return out2d.reshape(B, TOK, D)


# speedup vs baseline: 4.3939x; 3.7506x over previous
import jax
import jax.numpy as jnp
from jax import lax
from jax.experimental import pallas as pl
from jax.experimental.pallas import tpu as pltpu
from jax.experimental.pallas import tpu_sc as plsc

B = 16384
NF = 100
NC = 26
CARD = 1000
D = 128
TOK = NF + NC

NUM_CORES = 2
NUM_SUBCORES = 16
NW = NUM_CORES * NUM_SUBCORES
ROWS_PER_W = B * NC // NW
CHUNK_R = 128
N_CHUNKS = ROWS_PER_W // CHUNK_R


def _sc_gather_body(table_hbm, idx_hbm, out_hbm, idx_v, rows_v, sem):
    cid = lax.axis_index("c")
    sid = lax.axis_index("s")
    wid = sid * NUM_CORES + cid
    base_r = wid * ROWS_PER_W

    pltpu.sync_copy(idx_hbm.at[wid], idx_v)

    def chunk(g, carry):
        pltpu.async_copy(table_hbm.at[idx_v.at[g]], rows_v, sem).wait()
        pltpu.sync_copy(rows_v, out_hbm.at[pl.ds(base_r + g * CHUNK_R, CHUNK_R)])
        return carry

    lax.fori_loop(0, N_CHUNKS, chunk, 0)


def _sc_gather(emb_table, gidx):
    mesh = plsc.VectorSubcoreMesh(core_axis_name="c", subcore_axis_name="s")
    return pl.kernel(
        _sc_gather_body,
        out_type=jax.ShapeDtypeStruct((NC * B, D), jnp.float32),
        mesh=mesh,
        scratch_types=[
            pltpu.VMEM((N_CHUNKS, CHUNK_R), jnp.int32),
            pltpu.VMEM((CHUNK_R, D), jnp.float32),
            pltpu.SemaphoreType.DMA,
        ],
    )(emb_table, gidx)


BB = 128


def _tc_body(x_ref, w_ref, b_ref, cat_ref, cb_ref, out_ref):
    for f in range(NF):
        out_ref[f] = x_ref[:, f : f + 1] * w_ref[f : f + 1, :] + b_ref[f : f + 1, :]
    for f in range(NC):
        out_ref[NF + f] = cat_ref[f] + cb_ref[f : f + 1, :]


def _tc_assemble(x_num, num_weight, num_bias, cat3, cat_bias):
    return pl.pallas_call(
        _tc_body,
        grid=(B // BB,),
        in_specs=[
            pl.BlockSpec((BB, NF), lambda i: (i, 0)),
            pl.BlockSpec((NF, D), lambda i: (0, 0)),
            pl.BlockSpec((NF, D), lambda i: (0, 0)),
            pl.BlockSpec((NC, BB, D), lambda i: (0, i, 0)),
            pl.BlockSpec((NC, D), lambda i: (0, 0)),
        ],
        out_specs=pl.BlockSpec((TOK, BB, D), lambda i: (0, i, 0)),
        out_shape=jax.ShapeDtypeStruct((TOK, B, D), jnp.float32),
    )(x_num, num_weight, num_bias, cat3, cat_bias)


def kernel(x_num, x_cat, num_weight, num_bias, emb_table, cat_bias):
    offsets = jnp.arange(NC, dtype=jnp.int32) * CARD
    gidx = (x_cat.astype(jnp.int32).T + offsets[:, None]).reshape(
        NW, N_CHUNKS, CHUNK_R
    )
    cat_rows = _sc_gather(emb_table, gidx)
    out_t = _tc_assemble(
        x_num, num_weight, num_bias, cat_rows.reshape(NC, B, D), cat_bias
    )
    return out_t.transpose(1, 0, 2)


# speedup vs baseline: 4.6559x; 1.0596x over previous
import jax
import jax.numpy as jnp
from jax import lax
from jax.experimental import pallas as pl
from jax.experimental.pallas import tpu as pltpu
from jax.experimental.pallas import tpu_sc as plsc

B = 16384
NF = 100
NC = 26
CARD = 1000
D = 128
TOK = NF + NC

NUM_CORES = 2
NUM_SUBCORES = 16
NW = NUM_CORES * NUM_SUBCORES
ROWS_PER_W = B * NC // NW
CHUNK_R = 128
N_CHUNKS = ROWS_PER_W // CHUNK_R


def _sc_gather_body(table_hbm, idx_hbm, out_hbm, idx_v, rows_v, sem):
    cid = lax.axis_index("c")
    sid = lax.axis_index("s")
    wid = sid * NUM_CORES + cid
    base_r = wid * ROWS_PER_W

    pltpu.sync_copy(idx_hbm.at[wid], idx_v)

    def chunk(g, carry):
        pltpu.async_copy(table_hbm.at[idx_v.at[g]], rows_v, sem).wait()
        pltpu.sync_copy(rows_v, out_hbm.at[pl.ds(base_r + g * CHUNK_R, CHUNK_R)])
        return carry

    lax.fori_loop(0, N_CHUNKS, chunk, 0)


def _sc_gather(emb_table, gidx):
    mesh = plsc.VectorSubcoreMesh(core_axis_name="c", subcore_axis_name="s")
    return pl.kernel(
        _sc_gather_body,
        out_type=jax.ShapeDtypeStruct((NC * B, D), jnp.float32),
        mesh=mesh,
        scratch_types=[
            pltpu.VMEM((N_CHUNKS, CHUNK_R), jnp.int32),
            pltpu.VMEM((CHUNK_R, D), jnp.float32),
            pltpu.SemaphoreType.DMA,
        ],
    )(emb_table, gidx)


BB = 128
RB = 8192


def _tc_num_body(x_ref, w_ref, b_ref, out_ref):
    for f in range(NF):
        out_ref[f] = x_ref[:, f : f + 1] * w_ref[f : f + 1, :] + b_ref[f : f + 1, :]


def _tc_num(x_num, num_weight, num_bias):
    return pl.pallas_call(
        _tc_num_body,
        grid=(B // BB,),
        in_specs=[
            pl.BlockSpec((BB, NF), lambda i: (i, 0)),
            pl.BlockSpec((NF, D), lambda i: (0, 0)),
            pl.BlockSpec((NF, D), lambda i: (0, 0)),
        ],
        out_specs=pl.BlockSpec((NF, BB, D), lambda i: (0, i, 0)),
        out_shape=jax.ShapeDtypeStruct((TOK, B, D), jnp.float32),
    )(x_num, num_weight, num_bias)


def _tc_cat_body(cat_ref, cb_ref, prev_ref, out_ref):
    del prev_ref
    out_ref[...] = cat_ref[...] + cb_ref[0]


def _tc_cat_merge(cat_rows, cat_bias, prev_rows):
    return pl.pallas_call(
        _tc_cat_body,
        grid=(NC * B // RB,),
        in_specs=[
            pl.BlockSpec((RB, D), lambda j: (j, 0)),
            pl.BlockSpec((1, 1, D), lambda j: (j * RB // B, 0, 0)),
            pl.BlockSpec(memory_space=pl.ANY),
        ],
        out_specs=pl.BlockSpec((RB, D), lambda j: (NF * B // RB + j, 0)),
        out_shape=jax.ShapeDtypeStruct((TOK * B, D), jnp.float32),
        input_output_aliases={2: 0},
    )(cat_rows, cat_bias.reshape(NC, 1, D), prev_rows)


def kernel(x_num, x_cat, num_weight, num_bias, emb_table, cat_bias):
    offsets = jnp.arange(NC, dtype=jnp.int32) * CARD
    gidx = (x_cat.astype(jnp.int32).T + offsets[:, None]).reshape(
        NW, N_CHUNKS, CHUNK_R
    )
    cat_rows = _sc_gather(emb_table, gidx)
    num_t = _tc_num(x_num, num_weight, num_bias)
    out_rows = _tc_cat_merge(cat_rows, cat_bias, num_t.reshape(TOK * B, D))
    return out_rows.reshape(TOK, B, D).transpose(1, 0, 2)


# speedup vs baseline: 5.5046x; 1.1823x over previous
import jax
import jax.numpy as jnp
from jax import lax
from jax.experimental import pallas as pl
from jax.experimental.pallas import tpu as pltpu
from jax.experimental.pallas import tpu_sc as plsc

B = 16384
NF = 100
NC = 26
CARD = 1000
D = 128
TOK = NF + NC

NUM_CORES = 2
NUM_SUBCORES = 16
NW = NUM_CORES * NUM_SUBCORES
ROWS_PER_W = B * NC // NW
CHUNK_R = 128
N_CHUNKS = ROWS_PER_W // CHUNK_R
CAT_BASE = NF * B


def _sc_body(table_hbm, idx_hbm, bias_hbm, out_hbm,
             idx_v, bias_v, rows0, rows1, sem0, sem1):
    cid = lax.axis_index("c")
    sid = lax.axis_index("s")
    wid = sid * NUM_CORES + cid
    base_r = wid * ROWS_PER_W

    pltpu.sync_copy(idx_hbm.at[wid], idx_v)
    pltpu.sync_copy(bias_hbm, bias_v)

    def bias_add_and_flush(rows, g):
        f = (base_r + g * CHUNK_R) // B
        bvs = [bias_v[f, pl.ds(v * 16, 16)] for v in range(8)]

        def radd(r, c):
            for v in range(8):
                sl = pl.ds(v * 16, 16)
                rows[r, sl] = rows[r, sl] + bvs[v]
            return c

        lax.fori_loop(0, CHUNK_R, radd, 0)
        pltpu.sync_copy(
            rows, out_hbm.at[pl.ds(CAT_BASE + base_r + g * CHUNK_R, CHUNK_R)]
        )

    pltpu.async_copy(table_hbm.at[idx_v.at[0]], rows0, sem0)

    def pair(p, carry):
        g0 = 2 * p
        g1 = g0 + 1
        pltpu.make_async_copy(table_hbm.at[idx_v.at[g0]], rows0, sem0).wait()
        pltpu.async_copy(table_hbm.at[idx_v.at[g1]], rows1, sem1)
        bias_add_and_flush(rows0, g0)
        pltpu.make_async_copy(table_hbm.at[idx_v.at[g1]], rows1, sem1).wait()

        @pl.when(g1 + 1 < N_CHUNKS)
        def _():
            pltpu.async_copy(table_hbm.at[idx_v.at[g1 + 1]], rows0, sem0)

        bias_add_and_flush(rows1, g1)
        return carry

    lax.fori_loop(0, N_CHUNKS // 2, pair, 0)


def _sc_gather(emb_table, gidx, cat_bias):
    mesh = plsc.VectorSubcoreMesh(core_axis_name="c", subcore_axis_name="s")
    return pl.kernel(
        _sc_body,
        out_type=jax.ShapeDtypeStruct((TOK * B, D), jnp.float32),
        mesh=mesh,
        scratch_types=[
            pltpu.VMEM((N_CHUNKS, CHUNK_R), jnp.int32),
            pltpu.VMEM((NC, D), jnp.float32),
            pltpu.VMEM((CHUNK_R, D), jnp.float32),
            pltpu.VMEM((CHUNK_R, D), jnp.float32),
            pltpu.SemaphoreType.DMA,
            pltpu.SemaphoreType.DMA,
        ],
    )(emb_table, gidx, cat_bias)


BB = 128


def _tc_num_body(x_ref, w_ref, b_ref, prev_ref, out_ref):
    del prev_ref
    for f in range(NF):
        out_ref[f] = x_ref[:, f : f + 1] * w_ref[f : f + 1, :] + b_ref[f : f + 1, :]


def _tc_num(x_num, num_weight, num_bias, prev):
    return pl.pallas_call(
        _tc_num_body,
        grid=(B // BB,),
        in_specs=[
            pl.BlockSpec((BB, NF), lambda i: (i, 0)),
            pl.BlockSpec((NF, D), lambda i: (0, 0)),
            pl.BlockSpec((NF, D), lambda i: (0, 0)),
            pl.BlockSpec(memory_space=pl.ANY),
        ],
        out_specs=pl.BlockSpec((NF, BB, D), lambda i: (0, i, 0)),
        out_shape=jax.ShapeDtypeStruct((TOK, B, D), jnp.float32),
        input_output_aliases={3: 0},
    )(x_num, num_weight, num_bias, prev)


def kernel(x_num, x_cat, num_weight, num_bias, emb_table, cat_bias):
    offsets = jnp.arange(NC, dtype=jnp.int32) * CARD
    gidx = (x_cat.astype(jnp.int32).T + offsets[:, None]).reshape(
        NW, N_CHUNKS, CHUNK_R
    )
    rows = _sc_gather(emb_table, gidx, cat_bias)
    out_t = _tc_num(x_num, num_weight, num_bias, rows.reshape(TOK, B, D))
    return out_t.transpose(1, 0, 2)
